# baseline (speedup 1.0000x reference)
import jax
import jax.numpy as jnp
from jax.experimental import pallas as pl
from jax.experimental.pallas import tpu as pltpu

HID = 4096
NE = 64
CH = 1024
NBUF = 2


def _router_body(x_hbm, w_ref, b_ref, o_hbm, xbuf, obuf, insem, outsem):
    w = w_ref[...]
    bb = b_ref[...]
    nch = x_hbm.shape[0] // CH

    def read(i):
        return pltpu.make_async_copy(
            x_hbm.at[pl.ds(i * CH, CH)], xbuf.at[i % NBUF], insem.at[i % NBUF]
        )

    for i in range(min(NBUF, nch)):
        read(i).start()

    for i in range(nch):
        read(i).wait()
        x = xbuf[i % NBUF]
        logits = jax.lax.dot_general(
            x, w, (((1,), (1,)), ((), ())),
            preferred_element_type=jnp.float32,
        ) + bb
        m = jnp.max(logits, axis=-1, keepdims=True)
        e = jnp.exp(logits - m)
        res = e / jnp.sum(e, axis=-1, keepdims=True)
        if i + NBUF < nch:
            read(i + NBUF).start()
        obuf[pl.ds(i * CH, CH), :] = res

    pltpu.make_async_copy(obuf, o_hbm, outsem).start()
    pltpu.make_async_copy(obuf, o_hbm, outsem).wait()


def kernel(x, W, b):
    tokens = x.shape[0]
    return pl.pallas_call(
        _router_body,
        in_specs=[
            pl.BlockSpec(memory_space=pl.ANY),
            pl.BlockSpec((NE, HID), lambda: (0, 0)),
            pl.BlockSpec((1, NE), lambda: (0, 0)),
        ],
        out_specs=pl.BlockSpec(memory_space=pl.ANY),
        out_shape=jax.ShapeDtypeStruct((tokens, NE), jnp.float32),
        scratch_shapes=[
            pltpu.VMEM((NBUF, CH, HID), jnp.float32),
            pltpu.VMEM((tokens, NE), jnp.float32),
            pltpu.SemaphoreType.DMA((NBUF,)),
            pltpu.SemaphoreType.DMA,
        ],
    )(x, W, b.reshape(1, NE))


# speedup vs baseline: 1.0201x; 1.0201x over previous
import jax
import jax.numpy as jnp
from jax.experimental import pallas as pl
from jax.experimental.pallas import tpu as pltpu

HID = 4096
NE = 64
BT = 1024


def _router_body(x_ref, w_ref, b_ref, o_ref):
    i = pl.program_id(0)
    x = x_ref[...]
    w = w_ref[...]
    logits = jax.lax.dot_general(
        x, w, (((1,), (1,)), ((), ())), preferred_element_type=jnp.float32
    )
    logits = logits + b_ref[...]
    m = jnp.max(logits, axis=-1, keepdims=True)
    e = jnp.exp(logits - m)
    o_ref[pl.ds(i * BT, BT), :] = e / jnp.sum(e, axis=-1, keepdims=True)


def kernel(x, W, b):
    tokens = x.shape[0]
    return pl.pallas_call(
        _router_body,
        grid=(tokens // BT,),
        in_specs=[
            pl.BlockSpec((BT, HID), lambda i: (i, 0)),
            pl.BlockSpec((NE, HID), lambda i: (0, 0)),
            pl.BlockSpec((1, NE), lambda i: (0, 0)),
        ],
        out_specs=pl.BlockSpec((tokens, NE), lambda i: (0, 0)),
        out_shape=jax.ShapeDtypeStruct((tokens, NE), jnp.float32),
    )(x, W, b.reshape(1, NE))


# speedup vs baseline: 1.0305x; 1.0102x over previous
import jax
import jax.numpy as jnp
from jax.experimental import pallas as pl
from jax.experimental.pallas import tpu as pltpu

HID = 4096
NE = 64
BT = 1024


def _router_body(x_ref, w_ref, b_ref, o_ref):
    x = x_ref[...]
    w = w_ref[...]
    logits = jax.lax.dot_general(
        x, w, (((1,), (1,)), ((), ())), preferred_element_type=jnp.float32
    )
    logits = logits + b_ref[...]
    m = jnp.max(logits, axis=-1, keepdims=True)
    e = jnp.exp(logits - m)
    o_ref[...] = e / jnp.sum(e, axis=-1, keepdims=True)


def kernel(x, W, b):
    tokens = x.shape[0]
    return pl.pallas_call(
        _router_body,
        grid=(tokens // BT,),
        in_specs=[
            pl.BlockSpec((BT, HID), lambda i: (i, 0)),
            pl.BlockSpec((NE, HID), lambda i: (0, 0)),
            pl.BlockSpec((1, NE), lambda i: (0, 0)),
        ],
        out_specs=pl.BlockSpec((BT, NE), lambda i: (i, 0)),
        out_shape=jax.ShapeDtypeStruct((tokens, NE), jnp.float32),
        compiler_params=pltpu.CompilerParams(
            dimension_semantics=("parallel",),
        ),
    )(x, W, b.reshape(1, NE))
